# Initial kernel scaffold; baseline (speedup 1.0000x reference)
import jax
import jax.numpy as jnp
from jax.experimental import pallas as pl


def kernel(x0, x1, x2, up_index0, up_index1, bnd1_src, bnd1_dst, bnd2_src, bnd2_dst, batch0, batch1, batch2, W, b):
    raise NotImplementedError("write your pallas kernel here")


# speedup vs baseline: 5.3896x; 5.3896x over previous
import functools

import jax
import jax.numpy as jnp
from jax import lax
from jax.experimental import pallas as pl
from jax.experimental.pallas import tpu as pltpu
from jax.experimental.pallas import tpu_sc as plsc

NC = 2
NS = 16
L = 16
FP = 16
NUM_LAYERS = 3
NUM_GRAPHS = 512


def _ceil_to(x, m):
    return -(-x // m) * m


def _blocks(per_tile, blk):
    out, off = [], 0
    while off < per_tile:
        s = min(blk, per_tile - off)
        out.append((off, s))
        off += s
    return out


def _stripe(h):
    main = _ceil_to(-(-h // NS), 8)
    last = h - (NS - 1) * main
    assert 0 < last <= main, (h, main, last)
    return main, last


def _proj_body(x_ref, w_ref, o_ref):
    o_ref[...] = jnp.dot(x_ref[...], w_ref[...],
                         preferred_element_type=jnp.float32)


def _project(x, wt):
    n, f = x.shape
    blk = 2000
    while n % blk:
        blk //= 2
    return pl.pallas_call(
        _proj_body,
        grid=(n // blk,),
        in_specs=[
            pl.BlockSpec((blk, f), lambda i: (i, 0)),
            pl.BlockSpec((f, FP), lambda i: (0, 0)),
        ],
        out_specs=pl.BlockSpec((blk, FP), lambda i: (i, 0)),
        out_shape=jax.ShapeDtypeStruct((n, FP), jnp.float32),
    )(x, wt)


@functools.lru_cache(maxsize=None)
def _make_sc_kernel(ns, dst_levels, rels, blk, pool):
    h = {lvl: ns[lvl] // NC for lvl in range(3)}
    edge_sizes = sorted({s for (_, _, e) in rels
                         for (_, s) in _blocks(e // NS, blk)}, reverse=True)
    pool_sizes = ()
    if pool:
        psz = set()
        for lvl in dst_levels:
            for sz in _stripe(h[lvl]):
                for (_, c) in _blocks(sz, blk):
                    psz.add(c)
        pool_sizes = tuple(sorted(psz, reverse=True))

    scratch = [pltpu.VMEM_SHARED((h[lvl] + 8, FP), jnp.float32)
               for lvl in dst_levels]
    if pool:
        scratch.append(pltpu.VMEM_SHARED((NUM_GRAPHS, FP), jnp.float32))
    scratch.append(pltpu.VMEM((blk, FP), jnp.float32))
    scratch.append(pltpu.VMEM((blk,), jnp.int32))
    for s in edge_sizes:
        scratch.append(pltpu.VMEM((s,), jnp.int32))
    for s in edge_sizes:
        scratch.append(pltpu.VMEM((s,), jnp.int32))
    for s in pool_sizes:
        scratch.append(pltpu.VMEM((s,), jnp.int32))
    scratch.append(pltpu.SemaphoreType.DMA)

    if pool:
        out_type = jax.ShapeDtypeStruct((NC * NUM_GRAPHS, FP), jnp.float32)
    else:
        out_type = tuple(jax.ShapeDtypeStruct((ns[lvl], FP), jnp.float32)
                         for lvl in dst_levels)

    def body(*refs):
        ys = refs[0:3]
        k = 3
        eref = refs[k:k + 2 * len(rels)]
        k += 2 * len(rels)
        if pool:
            batches = dict(zip(dst_levels, refs[k:k + len(dst_levels)]))
            k += len(dst_levels)
            outs = {None: refs[k]}
            k += 1
        else:
            outs = dict(zip(dst_levels, refs[k:k + len(dst_levels)]))
            k += len(dst_levels)
        accs = dict(zip(dst_levels, refs[k:k + len(dst_levels)]))
        k += len(dst_levels)
        if pool:
            pool_acc = refs[k]
            k += 1
        rows = refs[k]
        dstbuf = refs[k + 1]
        k += 2
        srcbuf = dict(zip(edge_sizes, refs[k:k + len(edge_sizes)]))
        k += len(edge_sizes)
        adjbuf = dict(zip(edge_sizes, refs[k:k + len(edge_sizes)]))
        k += len(edge_sizes)
        pbuf = dict(zip(pool_sizes, refs[k:k + len(pool_sizes)]))
        k += len(pool_sizes)
        sem = refs[k]

        cid = lax.axis_index("c")
        sid = lax.axis_index("s")
        base = {lvl: cid * h[lvl] for lvl in dst_levels}

        def each_stripe(lvl, fn):
            main, last = _stripe(h[lvl])
            if last == main:
                fn(sid * main, main)
            else:
                @pl.when(sid < NS - 1)
                def _():
                    fn(sid * main, main)

                @pl.when(sid == NS - 1)
                def _():
                    fn((NS - 1) * main, last)

        for lvl in dst_levels:
            def seed(loc, size, lvl=lvl):
                pltpu.sync_copy(ys[lvl].at[pl.ds(base[lvl] + loc, size)],
                                accs[lvl].at[pl.ds(loc, size)])
            each_stripe(lvl, seed)
        if pool:
            @pl.when(sid == 0)
            def _():
                zv = jnp.zeros((L,), jnp.float32)

                def zb(i, c):
                    rows[i] = zv
                    return c
                lax.fori_loop(0, NUM_GRAPHS, zb, 0)
                pltpu.sync_copy(rows.at[pl.ds(0, NUM_GRAPHS)], pool_acc)
        plsc.subcore_barrier()

        for i, (sl, dl, e) in enumerate(rels):
            sref, dref = eref[2 * i], eref[2 * i + 1]
            pt = e // NS
            tb = sid * pt
            hd = h[dl]
            bd = base[dl]
            for (off, size) in _blocks(pt, blk):
                sb = srcbuf[size]
                ab = adjbuf[size]
                pltpu.sync_copy(sref.at[pl.ds(tb + off, size)], sb)
                pltpu.sync_copy(dref.at[pl.ds(tb + off, size)],
                                dstbuf.at[pl.ds(0, size)])
                pltpu.async_copy(ys[sl].at[sb], rows.at[pl.ds(0, size)],
                                 sem).wait()

                def adj_body(j, c, hd=hd, bd=bd, ab=ab):
                    d = dstbuf[pl.ds(j * L, L)] - bd
                    ok = (d >= 0) & (d < hd)
                    ab[pl.ds(j * L, L)] = jnp.where(ok, d, hd)
                    return c
                lax.fori_loop(0, size // L, adj_body, 0)
                pltpu.sync_copy(rows.at[pl.ds(0, size)], accs[dl].at[ab],
                                add=True)
        plsc.subcore_barrier()

        if not pool:
            for lvl in dst_levels:
                def wr(loc, size, lvl=lvl):
                    pltpu.sync_copy(accs[lvl].at[pl.ds(loc, size)],
                                    outs[lvl].at[pl.ds(base[lvl] + loc, size)])
                each_stripe(lvl, wr)
        else:
            for lvl in dst_levels:
                def pchunk(loc, size, lvl=lvl):
                    for (coff, csz) in _blocks(size, blk):
                        pb = pbuf[csz]
                        pltpu.sync_copy(accs[lvl].at[pl.ds(loc + coff, csz)],
                                        rows.at[pl.ds(0, csz)])
                        pltpu.sync_copy(
                            batches[lvl].at[pl.ds(base[lvl] + loc + coff, csz)],
                            pb)
                        pltpu.sync_copy(rows.at[pl.ds(0, csz)],
                                        pool_acc.at[pb], add=True)
                each_stripe(lvl, pchunk)
            plsc.subcore_barrier()

            @pl.when(sid == 0)
            def _():
                pltpu.sync_copy(
                    pool_acc,
                    outs[None].at[pl.ds(cid * NUM_GRAPHS, NUM_GRAPHS)])

    return pl.kernel(
        body,
        out_type=out_type,
        mesh=plsc.VectorSubcoreMesh(core_axis_name="c", subcore_axis_name="s",
                                    num_cores=NC, num_subcores=NS),
        scratch_types=scratch,
        compiler_params=pltpu.CompilerParams(use_tc_tiling_on_sc=False),
    )


def _pad_edges(src, dst, dump):
    e = src.shape[0]
    ep = _ceil_to(e, NS * L)
    if ep != e:
        src = jnp.concatenate([src, jnp.zeros((ep - e,), jnp.int32)])
        dst = jnp.concatenate([dst, jnp.full((ep - e,), dump, jnp.int32)])
    return src, dst


def kernel(x0, x1, x2, up_index0, up_index1, bnd1_src, bnd1_dst,
           bnd2_src, bnd2_dst, batch0, batch1, batch2, W, b):
    n0, n1, n2 = x0.shape[0], x1.shape[0], x2.shape[0]
    ns = (n0, n1, n2)
    wt = jnp.pad(W.T.astype(jnp.float32), ((0, 0), (0, FP - W.shape[0])))
    y0 = _project(x0, wt)
    y1 = _project(x1, wt)
    y2 = _project(x2, wt)

    i32 = jnp.int32
    s_up0, d_up0 = _pad_edges(up_index0[0].astype(i32),
                              up_index0[1].astype(i32), n0)
    s_b1, d_b1 = _pad_edges(bnd1_src.astype(i32), bnd1_dst.astype(i32), n1)
    s_up1, d_up1 = _pad_edges(up_index1[0].astype(i32),
                              up_index1[1].astype(i32), n1)
    s_b2, d_b2 = _pad_edges(bnd2_src.astype(i32), bnd2_dst.astype(i32), n2)

    edges_x = (s_up0, d_up0, s_b2, d_b2)
    rels_x = ((0, 0, s_up0.shape[0]), (1, 2, s_b2.shape[0]))
    edges_y = (s_b1, d_b1, s_up1, d_up1)
    rels_y = ((0, 1, s_b1.shape[0]), (1, 1, s_up1.shape[0]))

    kx = _make_sc_kernel(ns, (0, 2), rels_x, 4000, False)
    ky = _make_sc_kernel(ns, (1,), rels_y, 2000, False)
    kxp = _make_sc_kernel(ns, (0, 2), rels_x, 4000, True)
    kyp = _make_sc_kernel(ns, (1,), rels_y, 2000, True)

    for _ in range(NUM_LAYERS - 1):
        o0, o2 = kx(y0, y1, y2, *edges_x)
        (o1,) = ky(y0, y1, y2, *edges_y)
        y0, y1, y2 = o0, o1, o2
    b0 = batch0.astype(i32)
    b1 = batch1.astype(i32)
    b2 = batch2.astype(i32)
    px = kxp(y0, y1, y2, *edges_x, b0, b2)
    py = kyp(y0, y1, y2, *edges_y, b1)
    pooled = (px[:NUM_GRAPHS] + px[NUM_GRAPHS:]
              + py[:NUM_GRAPHS] + py[NUM_GRAPHS:])
    return pooled[:, :W.shape[0]] + b
